# Initial kernel scaffold; baseline (speedup 1.0000x reference)
import jax
import jax.numpy as jnp
from jax.experimental import pallas as pl


def kernel(x, n_objects, params):
    raise NotImplementedError("write your pallas kernel here")


# speedup vs baseline: 1.8413x; 1.8413x over previous
import functools

import numpy as np
import jax
import jax.numpy as jnp
from jax import lax
from jax.experimental import pallas as pl
from jax.experimental.pallas import tpu as pltpu
from jax.experimental.pallas import tpu_sc as plsc

_B = 8
_N = 128
_FEAT = 32
_HID = 256
_K = 32
_HEADS = 4
_DH = _HID // _HEADS
_P = _N * (_N - 1) // 2
_PPAD = 8192
_NBLK = 16
_BP = _PPAD // _NBLK
_CHUNKS = _P // 16
_NEG = np.float32(-3.0e38)

_SQ = _N * _N
_BI = 8
_NIB = _N // _BI

_row_np, _col_np = np.triu_indices(_N, k=1)
_ROW = _row_np.astype(np.int32)
_COL = _col_np.astype(np.int32)
_PIDX = (_ROW * _N + _COL).astype(np.int32)


def _enc_body(x_ref, w1, b1, w2, b2, nw1, nb1, nw2, nb2, hid_o, npj_o):
    h = jnp.maximum(x_ref[...] @ w1[...] + b1[...], 0.0)
    hid = h @ w2[...] + b2[...]
    hid_o[...] = hid
    n1 = jnp.maximum(hid @ nw1[...] + nb1[...], 0.0)
    npj_o[...] = n1 @ nw2[...] + nb2[...]


def _enc_call(x2, enc, npp):
    out = pl.pallas_call(
        _enc_body,
        out_shape=(
            jax.ShapeDtypeStruct((_B * _N, _HID), jnp.float32),
            jax.ShapeDtypeStruct((_B * _N, _HID), jnp.float32),
        ),
    )(x2, enc["W1"], enc["b1"].reshape(1, _HID), enc["W2"],
      enc["b2"].reshape(1, _HID),
      npp["W1"], npp["b1"].reshape(1, _HID), npp["W2"],
      npp["b2"].reshape(1, _HID))
    return out


def _edge_body(hi_ref, hf_ref, ew1, eb1, ew2, eb2, cw1, cb1, cw2, cb2_s,
               rep_o, pred_o):
    hi = hi_ref[...]
    hf = hf_ref[...]
    pair = (hi[:, None, :] + hf[None, :, :]).reshape(_BI * _N, _HID)
    h1 = jnp.maximum(pair @ ew1[...] + eb1[...], 0.0)
    rep = h1 @ ew2[...] + eb2[...]
    rep_o[...] = rep[None, :, :]
    c1 = jnp.maximum(rep @ cw1[...] + cb1[...], 0.0)
    pv = c1 @ cw2[...] + cb2_s[0]
    pred_o[...] = pv[None, :, :]


def _edge_call(hidden, en, ec):
    const2 = lambda b, i: (0, 0)
    rep, pred = pl.pallas_call(
        _edge_body,
        grid=(_B, _NIB),
        in_specs=[
            pl.BlockSpec((_BI, _HID), lambda b, i: (b * _NIB + i, 0)),
            pl.BlockSpec((_N, _HID), lambda b, i: (b, 0)),
            pl.BlockSpec((_HID, _HID), const2),
            pl.BlockSpec((1, _HID), const2),
            pl.BlockSpec((_HID, _HID), const2),
            pl.BlockSpec((1, _HID), const2),
            pl.BlockSpec((_HID, _HID), const2),
            pl.BlockSpec((1, _HID), const2),
            pl.BlockSpec((_HID, 1), const2),
            pl.BlockSpec(memory_space=pltpu.SMEM),
        ],
        out_specs=(
            pl.BlockSpec((1, _BI * _N, _HID), lambda b, i: (b, i, 0)),
            pl.BlockSpec((1, _BI * _N, 1), lambda b, i: (b * _NIB + i, 0, 0)),
        ),
        out_shape=(
            jax.ShapeDtypeStruct((_B, _SQ, _HID), jnp.float32),
            jax.ShapeDtypeStruct((_B * _NIB, _BI * _N, 1), jnp.float32),
        ),
    )(hidden, hidden, en["W1"], en["b1"].reshape(1, _HID), en["W2"],
      en["b2"].reshape(1, _HID), ec["W1"], ec["b1"].reshape(1, _HID),
      ec["W2"], ec["b2"])
    return rep, pred


def _sc_body(pf_hbm, rep_hbm, pidx_hbm, row_hbm, col_hbm,
             ep_o, tv_o, mrow_o, mcol_o, mask_o, mnod_o,
             pf_v, pidx_v, vals_v, row_v, col_v, tv_v, ti_v, tia_v,
             mrow_v, mcol_v, mask_v, rows_v, tmpv_v, tmpi_v, sem):
    wid = lax.axis_index("s") * 2 + lax.axis_index("c")

    @pl.when(wid < _B)
    def _():
        b = wid
        pltpu.sync_copy(pf_hbm.at[b], pf_v)
        pltpu.sync_copy(pidx_hbm, pidx_v)
        pltpu.sync_copy(row_hbm, row_v)
        pltpu.sync_copy(col_hbm, col_v)

        def gather_tri(c, _):
            sl = pl.ds(c * 16, 16)
            vals_v[sl] = plsc.load_gather(pf_v, [pidx_v[sl]])
            return 0

        lax.fori_loop(0, _CHUNKS, gather_tri, 0, unroll=4)
        pltpu.sync_copy(vals_v, ep_o.at[b])
        lanes = lax.iota(jnp.int32, 16)
        lane0 = lanes == 0

        def round_body(k, _carry):
            def chunk(c, carry):
                bv, bi = carry
                v = vals_v[pl.ds(c * 16, 16)]
                idx = c * 16 + lanes
                p = v > bv
                return jnp.where(p, v, bv), jnp.where(p, idx, bi)

            bv, bi = lax.fori_loop(
                0, _CHUNKS, chunk,
                (jnp.full((16,), _NEG, jnp.float32),
                 jnp.zeros((16,), jnp.int32)),
                unroll=4)
            for d in (8, 4, 2, 1):
                tmpv_v[pl.ds(0, 16)] = bv
                tmpi_v[pl.ds(0, 16)] = bi
                sh = jnp.bitwise_and(lanes + d, 15)
                gv = plsc.load_gather(tmpv_v, [sh])
                gi = plsc.load_gather(tmpi_v, [sh])
                t = (gv > bv) | ((gv == bv) & (gi < bi))
                bv = jnp.where(t, gv, bv)
                bi = jnp.where(t, gi, bi)
            kvec = jnp.full((16,), k, jnp.int32)
            plsc.store_scatter(tv_v, [kvec], bv, mask=lane0)
            plsc.store_scatter(ti_v, [kvec], bi, mask=lane0)
            plsc.store_scatter(vals_v, [bi],
                               jnp.full((16,), _NEG, jnp.float32), mask=lane0)
            return 0

        lax.fori_loop(0, _K, round_body, 0)

        for c in range(2):
            sl = pl.ds(c * 16, 16)
            iv = ti_v[sl]
            tia_v[sl] = plsc.load_gather(pidx_v, [iv]) + b * _SQ
            rg = plsc.load_gather(row_v, [iv])
            cg = plsc.load_gather(col_v, [iv])
            mk = tv_v[sl] > 0.0
            neg1 = jnp.full((16,), -1, jnp.int32)
            mrow_v[sl] = jnp.where(mk, rg, neg1)
            mcol_v[sl] = jnp.where(mk, cg, neg1)
            mask_v[sl] = mk.astype(jnp.int32)

        pltpu.async_copy(rep_hbm.at[tia_v], rows_v, sem).wait()
        pltpu.sync_copy(tv_v, tv_o.at[b])
        pltpu.sync_copy(mrow_v, mrow_o.at[b])
        pltpu.sync_copy(mcol_v, mcol_o.at[b])
        pltpu.sync_copy(mask_v, mask_o.at[b])
        pltpu.sync_copy(rows_v, mnod_o.at[b])


def _sc_topk_call(pred_full, rep2d, pidxlut, rowlut, collut):
    mesh = plsc.VectorSubcoreMesh(core_axis_name="c", subcore_axis_name="s")
    kern = functools.partial(
        pl.kernel,
        mesh=mesh,
        compiler_params=pltpu.CompilerParams(needs_layout_passes=False),
        out_type=[
            jax.ShapeDtypeStruct((_B, _P), jnp.float32),
            jax.ShapeDtypeStruct((_B, _K), jnp.float32),
            jax.ShapeDtypeStruct((_B, _K), jnp.int32),
            jax.ShapeDtypeStruct((_B, _K), jnp.int32),
            jax.ShapeDtypeStruct((_B, _K), jnp.int32),
            jax.ShapeDtypeStruct((_B, _K, _HID), jnp.float32),
        ],
        scratch_types=[
            pltpu.VMEM((_SQ,), jnp.float32),
            pltpu.VMEM((_P,), jnp.int32),
            pltpu.VMEM((_P,), jnp.float32),
            pltpu.VMEM((_P,), jnp.int32),
            pltpu.VMEM((_P,), jnp.int32),
            pltpu.VMEM((_K,), jnp.float32),
            pltpu.VMEM((_K,), jnp.int32),
            pltpu.VMEM((_K,), jnp.int32),
            pltpu.VMEM((_K,), jnp.int32),
            pltpu.VMEM((_K,), jnp.int32),
            pltpu.VMEM((_K,), jnp.int32),
            pltpu.VMEM((_K, _HID), jnp.float32),
            pltpu.VMEM((16,), jnp.float32),
            pltpu.VMEM((16,), jnp.int32),
            pltpu.SemaphoreType.DMA,
        ],
    )(_sc_body)
    return kern(pred_full, rep2d, pidxlut, rowlut, collut)


def _ln(x):
    m = jnp.mean(x, axis=-1, keepdims=True)
    v = jnp.mean((x - m) ** 2, axis=-1, keepdims=True)
    return (x - m) * lax.rsqrt(v + 1e-6)


def _softmax(x):
    mx = jnp.max(x, axis=-1, keepdims=True)
    e = jnp.exp(x - mx)
    return e / jnp.sum(e, axis=-1, keepdims=True)


def _dit_body(npj_ref, mnod_ref, tvc_ref, maskc_ref, maskr_ref, nsc_ref,
              mW1m, mw1l, mb1, mW2, mb2,
              l0, l1, Wout, bout, e2W1,
              u1n_o, u1m_o):
    b = pl.program_id(0)
    mnod = mnod_ref[...]
    tvc = tvc_ref[...].reshape(_K, 1)
    maskc = maskc_ref[...].reshape(_K, 1)
    maskr = maskr_ref[...].reshape(1, _K)

    meta_h = jnp.maximum(mnod @ mW1m[...] + tvc * mw1l[...] + mb1[...], 0.0)
    meta_proj = meta_h @ mW2[...] + mb2[...]
    merged = jnp.concatenate([npj_ref[...], meta_proj], axis=0)

    qf_col = jnp.concatenate([jnp.ones((_N, 1), jnp.float32), maskc], axis=0)
    qf_row = jnp.concatenate([jnp.ones((1, _N), jnp.float32), maskr], axis=1)
    mcnt = jnp.sum(maskc)
    cnt = jnp.float32(_N) + mcnt
    ctx = jnp.sum(merged * qf_col, axis=0, keepdims=True) / cnt
    nsc = nsc_ref[b]
    nmeta = mcnt / jnp.float32(_K)
    inval_pen = (1.0 - qf_row) * 1e9

    q = merged
    for (Wm, wn, wk, bmod, Wq, bq, Wk_, bk, Wv, bv, Wo, bo,
         Wm1, bm1, Wm2, bm2) in (l0, l1):
        mod = ctx @ Wm[...] + nsc * wn[...] + nmeta * wk[...] + bmod[...]
        s1 = mod[:, 0:_HID]
        sc1 = mod[:, _HID:2 * _HID]
        g1 = mod[:, 2 * _HID:3 * _HID]
        s2 = mod[:, 3 * _HID:4 * _HID]
        sc2 = mod[:, 4 * _HID:5 * _HID]
        g2 = mod[:, 5 * _HID:6 * _HID]

        h = _ln(q) * (1.0 + sc1) + s1
        qq = h @ Wq[...] + bq[...]
        kk = h @ Wk_[...] + bk[...]
        vv = h @ Wv[...] + bv[...]
        outs = []
        for hh in range(_HEADS):
            s = slice(hh * _DH, (hh + 1) * _DH)
            lg = lax.dot_general(qq[:, s], kk[:, s],
                                 (((1,), (1,)), ((), ()))) * (1.0 / np.sqrt(_DH))
            a = _softmax(lg - inval_pen)
            outs.append(a @ vv[:, s])
        att = jnp.concatenate(outs, axis=1) @ Wo[...] + bo[...]
        q = q + g1 * att
        h = _ln(q) * (1.0 + sc2) + s2
        m1 = jnp.maximum(h @ Wm1[...] + bm1[...], 0.0)
        q = q + g2 * (m1 @ Wm2[...] + bm2[...])

    upd = (_ln(q) @ Wout[...] + bout[...]) * qf_col
    u1 = upd @ e2W1[...]
    u1n_o[...] = u1[:_N]
    u1m_o[...] = u1[_N:]


def _dit_call(npj, mnod2, tvc, maskc, maskr, nsc, mp, layers, Wout, bout,
              e2W1):
    const2 = lambda b: (0, 0)

    def w2(shape):
        return pl.BlockSpec(shape, const2)

    layer_arrs = []
    layer_specs = []
    for lyr in layers:
        arrs = (lyr["Wmod"][:_HID], lyr["Wmod"][_HID:_HID + 1],
                lyr["Wmod"][_HID + 1:_HID + 2], lyr["bmod"].reshape(1, -1),
                lyr["Wq"], lyr["bq"].reshape(1, -1),
                lyr["Wk"], lyr["bk"].reshape(1, -1),
                lyr["Wv"], lyr["bv"].reshape(1, -1),
                lyr["Wo"], lyr["bo"].reshape(1, -1),
                lyr["Wm1"], lyr["bm1"].reshape(1, -1),
                lyr["Wm2"], lyr["bm2"].reshape(1, -1))
        layer_arrs.append(arrs)
        layer_specs.append(tuple(w2(a.shape) for a in arrs))

    in_specs = [
        pl.BlockSpec((_N, _HID), lambda b: (b, 0)),
        pl.BlockSpec((_K, _HID), lambda b: (b, 0)),
        pl.BlockSpec((1, _K, 1), lambda b: (b, 0, 0)),
        pl.BlockSpec((1, _K, 1), lambda b: (b, 0, 0)),
        pl.BlockSpec((1, 1, _K), lambda b: (b, 0, 0)),
        pl.BlockSpec(memory_space=pltpu.SMEM),
        w2((_HID, _HID)), w2((1, _HID)), w2((1, _HID)),
        w2((_HID, _HID)), w2((1, _HID)),
        layer_specs[0], layer_specs[1],
        w2((_HID, _HID)), w2((1, _HID)),
        w2((_HID, _HID)),
    ]
    u1n, u1m = pl.pallas_call(
        _dit_body,
        grid=(_B,),
        in_specs=in_specs,
        out_specs=(
            pl.BlockSpec((_N, _HID), lambda b: (b, 0)),
            pl.BlockSpec((_K, _HID), lambda b: (b, 0)),
        ),
        out_shape=(
            jax.ShapeDtypeStruct((_B * _N, _HID), jnp.float32),
            jax.ShapeDtypeStruct((_B * _K, _HID), jnp.float32),
        ),
    )(npj, mnod2, tvc, maskc, maskr, nsc,
      mp["W1"][:_HID], mp["W1"][_HID:_HID + 1], mp["b1"].reshape(1, -1),
      mp["W2"], mp["b2"].reshape(1, -1),
      layer_arrs[0], layer_arrs[1],
      Wout, bout.reshape(1, -1), e2W1)
    return u1n, u1m


def _edge2_body(u1n_ref, u1m_ref, eb1, ew2, eb2, cw1, cb1, cw2r, cb2_s,
                out_ref):
    un = u1n_ref[...]
    um = u1m_ref[...]
    for j in range(_K):
        h1 = jnp.maximum(un + um[j:j + 1] + eb1[...], 0.0)
        rep = h1 @ ew2[...] + eb2[...]
        c = jnp.maximum(rep @ cw1[...] + cb1[...], 0.0)
        pj = jnp.sum(c * cw2r[...], axis=1, keepdims=True) + cb2_s[0]
        out_ref[:, :, j:j + 1] = pj[None, :, :]


def _edge2_call(u1n, u1m, en2, ec2):
    const2 = lambda b: (0, 0)
    out = pl.pallas_call(
        _edge2_body,
        grid=(_B,),
        in_specs=[
            pl.BlockSpec((_N, _HID), lambda b: (b, 0)),
            pl.BlockSpec((_K, _HID), lambda b: (b, 0)),
            pl.BlockSpec((1, _HID), const2),
            pl.BlockSpec((_HID, _HID), const2),
            pl.BlockSpec((1, _HID), const2),
            pl.BlockSpec((_HID, _HID), const2),
            pl.BlockSpec((1, _HID), const2),
            pl.BlockSpec((1, _HID), const2),
            pl.BlockSpec(memory_space=pltpu.SMEM),
        ],
        out_specs=pl.BlockSpec((1, _N, _K), lambda b: (b, 0, 0)),
        out_shape=jax.ShapeDtypeStruct((_B, _N, _K), jnp.float32),
    )(u1n, u1m, en2["b1"].reshape(1, _HID), en2["W2"],
      en2["b2"].reshape(1, _HID), ec2["W1"], ec2["b1"].reshape(1, _HID),
      ec2["W2"].reshape(1, _HID), ec2["b2"])
    return out


def kernel(x, n_objects, params):
    x2 = x.reshape(_B * _N, _FEAT)
    pidxlut = jnp.asarray(_PIDX)
    rowlut = jnp.asarray(_ROW)
    collut = jnp.asarray(_COL)

    hidden, npj = _enc_call(x2, params["enc"], params["node_proj"])
    rep, pred = _edge_call(hidden, params["edge_net"], params["edge_cls"])
    pred_full = pred.reshape(_B, _SQ)

    edge_pred, tv, mrow, mcol, maskint, mnod = _sc_topk_call(
        pred_full, rep.reshape(_B * _SQ, _HID), pidxlut, rowlut, collut)

    prob_mask = maskint.astype(bool)
    meta_idx = jnp.stack([mrow, mcol], axis=-1)
    maskf = maskint.astype(jnp.float32)
    nsc = (n_objects.astype(jnp.float32) - 2.0) / 126.0

    u1n, u1m = _dit_call(
        npj, mnod.reshape(_B * _K, _HID), tv.reshape(_B, _K, 1),
        maskf.reshape(_B, _K, 1), maskf.reshape(_B, 1, _K), nsc,
        params["meta_node_proj"], params["dit"]["layers"],
        params["dit"]["Wout"], params["dit"]["bout"],
        params["edge_net2"]["W1"])

    edge_pred2 = _edge2_call(u1n, u1m, params["edge_net2"],
                             params["edge_cls2"])
    return edge_pred2, edge_pred, meta_idx, prob_mask
